# baseline (speedup 1.0000x reference)
import jax
import jax.numpy as jnp
from jax.experimental import pallas as pl
from jax.experimental.pallas import tpu as pltpu


_BT = 512
_NBUF = 3


def _router_block(x_hbm, w_hbm, logits_ref, probs_ref, xbuf, wbuf, sems, wsem):
    i = pl.program_id(0)
    nsteps = pl.num_programs(0)

    def _start(step, slot):
        pltpu.make_async_copy(
            x_hbm.at[pl.ds(step * _BT, _BT), :],
            xbuf.at[slot],
            sems.at[slot],
        ).start()

    @pl.when(i == 0)
    def _warmup():
        pltpu.make_async_copy(w_hbm, wbuf, wsem).start()
        for b in range(_NBUF):
            _start(b, b)
        pltpu.make_async_copy(w_hbm, wbuf, wsem).wait()

    slot = jax.lax.rem(i, _NBUF)
    pltpu.make_async_copy(
        x_hbm.at[pl.ds(i * _BT, _BT), :], xbuf.at[slot], sems.at[slot]
    ).wait()

    logits = jax.lax.dot_general(
        xbuf[slot], wbuf[...], (((1,), (1,)), ((), ())),
        preferred_element_type=jnp.float32,
    )
    rows = pl.ds(i * _BT, _BT)
    logits_ref[rows, :] = logits
    m = jnp.max(logits, axis=-1, keepdims=True)
    e = jnp.exp(logits - m)
    probs_ref[rows, :] = e / jnp.sum(e, axis=-1, keepdims=True)

    @pl.when(i + _NBUF < nsteps)
    def _prefetch():
        _start(i + _NBUF, slot)


def kernel(x, W):
    tokens, dim = x.shape
    n_experts = W.shape[0]
    grid = (tokens // _BT,)
    logits, probs = pl.pallas_call(
        _router_block,
        grid=grid,
        in_specs=[
            pl.BlockSpec(memory_space=pl.ANY),
            pl.BlockSpec(memory_space=pl.ANY),
        ],
        out_specs=[
            pl.BlockSpec((tokens, n_experts), lambda i: (0, 0)),
            pl.BlockSpec((tokens, n_experts), lambda i: (0, 0)),
        ],
        out_shape=[
            jax.ShapeDtypeStruct((tokens, n_experts), jnp.float32),
            jax.ShapeDtypeStruct((tokens, n_experts), jnp.float32),
        ],
        scratch_shapes=[
            pltpu.VMEM((_NBUF, _BT, dim), jnp.float32),
            pltpu.VMEM((n_experts, dim), jnp.float32),
            pltpu.SemaphoreType.DMA((_NBUF,)),
            pltpu.SemaphoreType.DMA,
        ],
        compiler_params=pltpu.CompilerParams(
            dimension_semantics=("arbitrary",),
            vmem_limit_bytes=63 * 1024 * 1024,
        ),
    )(x, W)
    return logits, probs, probs


# speedup vs baseline: 1.0142x; 1.0142x over previous
import jax
import jax.numpy as jnp
from jax.experimental import pallas as pl
from jax.experimental.pallas import tpu as pltpu


_BT = 512
_NBUF = 6


def _router_block(x_hbm, w_hbm, logits_ref, probs_ref, xbuf, wbuf, sems, wsem):
    i = pl.program_id(0)
    nsteps = pl.num_programs(0)

    def _start(step, slot, prio=0):
        pltpu.make_async_copy(
            x_hbm.at[pl.ds(step * _BT, _BT), :],
            xbuf.at[slot],
            sems.at[slot],
        ).start(priority=prio)

    @pl.when(i == 0)
    def _warmup():
        pltpu.make_async_copy(w_hbm, wbuf, wsem).start()
        for b in range(_NBUF):
            _start(b, b, prio=b % 2)
        pltpu.make_async_copy(w_hbm, wbuf, wsem).wait()

    slot = jax.lax.rem(i, _NBUF)
    pltpu.make_async_copy(
        x_hbm.at[pl.ds(i * _BT, _BT), :], xbuf.at[slot], sems.at[slot]
    ).wait()

    logits = jax.lax.dot_general(
        xbuf[slot], wbuf[...], (((1,), (1,)), ((), ())),
        preferred_element_type=jnp.float32,
    )
    logits_ref[...] = logits
    m = jnp.max(logits, axis=-1, keepdims=True)
    e = jnp.exp(logits - m)
    probs_ref[...] = e / jnp.sum(e, axis=-1, keepdims=True)

    nxt = i + _NBUF
    even = jax.lax.rem(nxt, 2) == 0

    @pl.when(jnp.logical_and(nxt < nsteps, even))
    def _prefetch_even():
        _start(nxt, slot, prio=0)

    @pl.when(jnp.logical_and(nxt < nsteps, jnp.logical_not(even)))
    def _prefetch_odd():
        _start(nxt, slot, prio=1)


def kernel(x, W):
    tokens, dim = x.shape
    n_experts = W.shape[0]
    grid = (tokens // _BT,)
    logits, probs = pl.pallas_call(
        _router_block,
        grid=grid,
        in_specs=[
            pl.BlockSpec(memory_space=pl.ANY),
            pl.BlockSpec(memory_space=pl.ANY),
        ],
        out_specs=[
            pl.BlockSpec((_BT, n_experts), lambda i: (i, 0)),
            pl.BlockSpec((_BT, n_experts), lambda i: (i, 0)),
        ],
        out_shape=[
            jax.ShapeDtypeStruct((tokens, n_experts), jnp.float32),
            jax.ShapeDtypeStruct((tokens, n_experts), jnp.float32),
        ],
        scratch_shapes=[
            pltpu.VMEM((_NBUF, _BT, dim), jnp.float32),
            pltpu.VMEM((n_experts, dim), jnp.float32),
            pltpu.SemaphoreType.DMA((_NBUF,)),
            pltpu.SemaphoreType.DMA,
        ],
        compiler_params=pltpu.CompilerParams(
            dimension_semantics=("arbitrary",),
            vmem_limit_bytes=63 * 1024 * 1024,
        ),
    )(x, W)
    return logits, probs, probs
